# baseline (speedup 1.0000x reference)
import jax
import jax.numpy as jnp
from jax.experimental import pallas as pl
from jax.experimental.pallas import tpu as pltpu

NEG = -1e30
_B, _T, _V, _U = 16, 2048, 64, 256
_W = 384
_TB = 128
_NB = _T // _TB
_UF = 32
_TINY = 1e-37


def _ctc_kernel(logits_ref, targets_ref, loglen_ref, tgtlen_ref, out_ref,
                gf_scr, gb_scr, oh_scr):
    lane = jax.lax.broadcasted_iota(jnp.int32, (_B, _W), 1)

    tgt = targets_ref[:, :]
    pad_col = jnp.where(
        jax.lax.broadcasted_iota(jnp.int32, (_B, _W - _U), 1) == 0, 0, -1)
    tpad = jnp.concatenate([tgt, pad_col], axis=1)

    iota_v = jax.lax.broadcasted_iota(jnp.int32, (_V, _W), 0)
    for b in range(_B):
        row = jax.lax.broadcast_in_dim(tpad[b, :], (_V, _W), (1,))
        oh_scr[b] = (iota_v == row).astype(jnp.float32)

    prev = jnp.concatenate(
        [jnp.full((_B, 1), -1, jnp.int32), tpad[:, :_W - 1]], axis=1)
    skip_mul = jnp.where(tpad != prev, 1.0, 0.0).astype(jnp.float32)
    skip_l = jnp.concatenate(
        [skip_mul[:, 1:], jnp.zeros((_B, 1), jnp.float32)], axis=1)

    loglen = loglen_ref[:, :]
    tgtlen = tgtlen_ref[:, :]

    init_de = jnp.where(lane == tgtlen, 0.0, NEG)
    init_do = jnp.where(lane == tgtlen - 1, 0.0, NEG)

    def fill_block(blk, scr):
        t0 = blk * _TB
        for b in range(_B):
            a = logits_ref[b, pl.ds(t0, _TB), :]
            m = jnp.max(a, axis=1, keepdims=True)
            lse = jnp.log(jnp.sum(jnp.exp(a - m), axis=1, keepdims=True)) + m
            gb = jnp.dot(a, oh_scr[b], preferred_element_type=jnp.float32)
            scr[b] = gb - lse

    def read_g(scr, t_local):
        return scr[:, pl.ds(t_local, 1), :].reshape(_B, _W)

    def fstep(t_local, alpha_e, alpha_o):
        g_t = read_g(gf_scr, t_local)
        blank = jax.lax.broadcast_in_dim(g_t[:, _U], (_B, 1), (0,))
        shift_o = jnp.concatenate(
            [jnp.full((_B, 1), NEG, jnp.float32), alpha_o[:, :-1]], axis=1)
        m = jnp.maximum(jnp.maximum(alpha_o, alpha_e), shift_o)
        x_o = jnp.exp(alpha_o - m)
        x_e = jnp.exp(alpha_e - m)
        x_s = jnp.exp(shift_o - m)
        new_o = m + jnp.log(jnp.maximum(x_o + x_e + x_s * skip_mul,
                                        _TINY)) + g_t
        new_e = m + jnp.log(jnp.maximum(x_e + x_s, _TINY)) + blank
        return new_e, new_o

    def bstep(t_local, tt, delta_e, delta_o):
        g_t = read_g(gb_scr, t_local)
        blank = jax.lax.broadcast_in_dim(g_t[:, _U], (_B, 1), (0,))
        d_o = delta_o + g_t
        d_e = delta_e + blank
        sle = jnp.concatenate(
            [d_e[:, 1:], jnp.full((_B, 1), NEG, jnp.float32)], axis=1)
        slo = jnp.concatenate(
            [d_o[:, 1:], jnp.full((_B, 1), NEG, jnp.float32)], axis=1)
        m2 = jnp.maximum(d_e, d_o)
        new_e = m2 + jnp.log(jnp.maximum(
            jnp.exp(d_e - m2) + jnp.exp(d_o - m2), _TINY))
        m3 = jnp.maximum(jnp.maximum(d_o, sle), slo)
        new_o = m3 + jnp.log(jnp.maximum(
            jnp.exp(d_o - m3) + jnp.exp(sle - m3)
            + jnp.exp(slo - m3) * skip_l, _TINY))
        live = tt < loglen
        return (jnp.where(live, new_e, init_de),
                jnp.where(live, new_o, init_do))

    def both(j, t0b, c):
        ae, ao, de, do = c
        ae, ao = fstep(j, ae, ao)
        de, do = bstep(_TB - 1 - j, t0b - j, de, do)
        return ae, ao, de, do

    def run_block(t0b, carry, first):
        def inner(i, c):
            tl = first + i * _UF
            for k in range(_UF):
                c = both(tl + k, t0b, c)
            return c
        if first:
            for k in range(1, _UF):
                carry = both(k, t0b, carry)
            return jax.lax.fori_loop(0, _TB // _UF - 1, inner, carry)
        return jax.lax.fori_loop(0, _TB // _UF, inner, carry)

    fill_block(0, gf_scr)
    fill_block(_NB - 1, gb_scr)
    g0 = read_g(gf_scr, 0)
    blank0 = jax.lax.broadcast_in_dim(g0[:, _U], (_B, 1), (0,))
    alpha_e = jnp.where(lane == 0, jnp.broadcast_to(blank0, (_B, _W)), NEG)
    alpha_o = jnp.where(lane == 0, g0, NEG)
    delta_e, delta_o = bstep(_TB - 1, _T - 1, init_de, init_do)
    carry = run_block(_T - 1, (alpha_e, alpha_o, delta_e, delta_o), _UF)

    def block_body(k, c):
        fill_block(k, gf_scr)
        fill_block(_NB - 1 - k, gb_scr)
        return run_block(_T - 1 - k * _TB, c, 0)

    carry = jax.lax.fori_loop(1, _NB // 2, block_body, carry)
    alpha_e, alpha_o, delta_e, delta_o = carry

    s_e = alpha_e + delta_e
    s_o = alpha_o + delta_o
    m = jnp.maximum(jnp.max(s_e, axis=1, keepdims=True),
                    jnp.max(s_o, axis=1, keepdims=True))
    z = (jnp.sum(jnp.exp(s_e - m), axis=1, keepdims=True)
         + jnp.sum(jnp.exp(s_o - m), axis=1, keepdims=True))
    ll = m + jnp.log(z)
    out_ref[:, :] = jnp.broadcast_to(-ll, (_B, 128))


def _run(logits, targets, loglen, tgtlen):
    return pl.pallas_call(
        _ctc_kernel,
        out_shape=jax.ShapeDtypeStruct((_B, 128), jnp.float32),
        scratch_shapes=[
            pltpu.VMEM((_B, _TB, _W), jnp.float32),
            pltpu.VMEM((_B, _TB, _W), jnp.float32),
            pltpu.VMEM((_B, _V, _W), jnp.float32),
        ],
    )(logits, targets, loglen, tgtlen)


@jax.jit
def kernel(logits, targets, logits_lengths, targets_lengths):
    loglen = logits_lengths.astype(jnp.int32).reshape(_B, 1)
    tgtlen = targets_lengths.astype(jnp.int32).reshape(_B, 1)
    out = _run(logits, targets.astype(jnp.int32), loglen, tgtlen)
    return out[:, 0]


# speedup vs baseline: 1.0136x; 1.0136x over previous
import jax
import jax.numpy as jnp
from jax.experimental import pallas as pl
from jax.experimental.pallas import tpu as pltpu

NEG = -1e30
_B, _T, _V, _U = 16, 2048, 64, 256
_W = 384
_TB = 128
_NB = _T // _TB
_UF = 16
_TINY = 1e-37


def _ctc_kernel(logits_ref, targets_ref, loglen_ref, tgtlen_ref, out_ref,
                gf_scr, gb_scr, oh_scr):
    lane = jax.lax.broadcasted_iota(jnp.int32, (_B, _W), 1)

    tgt = targets_ref[:, :]
    pad_col = jnp.where(
        jax.lax.broadcasted_iota(jnp.int32, (_B, _W - _U), 1) == 0, 0, -1)
    tpad = jnp.concatenate([tgt, pad_col], axis=1)

    iota_v = jax.lax.broadcasted_iota(jnp.int32, (_V, _W), 0)
    for b in range(_B):
        row = jax.lax.broadcast_in_dim(tpad[b, :], (_V, _W), (1,))
        oh_scr[b] = (iota_v == row).astype(jnp.float32)

    prev = jnp.concatenate(
        [jnp.full((_B, 1), -1, jnp.int32), tpad[:, :_W - 1]], axis=1)
    skip_mul = jnp.where(tpad != prev, 1.0, 0.0).astype(jnp.float32)
    skip_l = jnp.concatenate(
        [skip_mul[:, 1:], jnp.zeros((_B, 1), jnp.float32)], axis=1)

    loglen = loglen_ref[:, :]
    tgtlen = tgtlen_ref[:, :]

    init_de = jnp.where(lane == tgtlen, 0.0, NEG)
    init_do = jnp.where(lane == tgtlen - 1, 0.0, NEG)

    def fill_block(blk, scr):
        t0 = blk * _TB
        for b in range(_B):
            a = logits_ref[b, pl.ds(t0, _TB), :]
            m = jnp.max(a, axis=1, keepdims=True)
            lse = jnp.log(jnp.sum(jnp.exp(a - m), axis=1, keepdims=True)) + m
            gb = jnp.dot(a, oh_scr[b], preferred_element_type=jnp.float32)
            scr[b] = gb - lse

    def read_g(scr, t_local):
        return scr[:, pl.ds(t_local, 1), :].reshape(_B, _W)

    def fstep(t_local, alpha_e, alpha_o):
        g_t = read_g(gf_scr, t_local)
        blank = jax.lax.broadcast_in_dim(g_t[:, _U], (_B, 1), (0,))
        shift_o = jnp.concatenate(
            [jnp.full((_B, 1), NEG, jnp.float32), alpha_o[:, :-1]], axis=1)
        m = jnp.maximum(jnp.maximum(alpha_o, alpha_e), shift_o)
        x_o = jnp.exp(alpha_o - m)
        x_e = jnp.exp(alpha_e - m)
        x_s = jnp.exp(shift_o - m)
        new_o = m + jnp.log(jnp.maximum(x_o + x_e + x_s * skip_mul,
                                        _TINY)) + g_t
        new_e = m + jnp.log(jnp.maximum(x_e + x_s, _TINY)) + blank
        return new_e, new_o

    def bstep(t_local, tt, delta_e, delta_o):
        g_t = read_g(gb_scr, t_local)
        blank = jax.lax.broadcast_in_dim(g_t[:, _U], (_B, 1), (0,))
        d_o = delta_o + g_t
        d_e = delta_e + blank
        sle = jnp.concatenate(
            [d_e[:, 1:], jnp.full((_B, 1), NEG, jnp.float32)], axis=1)
        slo = jnp.concatenate(
            [d_o[:, 1:], jnp.full((_B, 1), NEG, jnp.float32)], axis=1)
        m2 = jnp.maximum(d_e, d_o)
        new_e = m2 + jnp.log(jnp.maximum(
            jnp.exp(d_e - m2) + jnp.exp(d_o - m2), _TINY))
        m3 = jnp.maximum(jnp.maximum(d_o, sle), slo)
        new_o = m3 + jnp.log(jnp.maximum(
            jnp.exp(d_o - m3) + jnp.exp(sle - m3)
            + jnp.exp(slo - m3) * skip_l, _TINY))
        live = tt < loglen
        return (jnp.where(live, new_e, init_de),
                jnp.where(live, new_o, init_do))

    def both(j, t0b, c):
        ae, ao, de, do = c
        ae, ao = fstep(j, ae, ao)
        de, do = bstep(_TB - 1 - j, t0b - j, de, do)
        return ae, ao, de, do

    def run_block(t0b, carry, first):
        def inner(i, c):
            tl = first + i * _UF
            for k in range(_UF):
                c = both(tl + k, t0b, c)
            return c
        if first:
            for k in range(1, _UF):
                carry = both(k, t0b, carry)
            return jax.lax.fori_loop(0, _TB // _UF - 1, inner, carry)
        return jax.lax.fori_loop(0, _TB // _UF, inner, carry)

    fill_block(0, gf_scr)
    fill_block(_NB - 1, gb_scr)
    g0 = read_g(gf_scr, 0)
    blank0 = jax.lax.broadcast_in_dim(g0[:, _U], (_B, 1), (0,))
    alpha_e = jnp.where(lane == 0, jnp.broadcast_to(blank0, (_B, _W)), NEG)
    alpha_o = jnp.where(lane == 0, g0, NEG)
    delta_e, delta_o = bstep(_TB - 1, _T - 1, init_de, init_do)
    carry = run_block(_T - 1, (alpha_e, alpha_o, delta_e, delta_o), _UF)

    def block_body(k, c):
        fill_block(k, gf_scr)
        fill_block(_NB - 1 - k, gb_scr)
        return run_block(_T - 1 - k * _TB, c, 0)

    carry = jax.lax.fori_loop(1, _NB // 2, block_body, carry)
    alpha_e, alpha_o, delta_e, delta_o = carry

    s_e = alpha_e + delta_e
    s_o = alpha_o + delta_o
    m = jnp.maximum(jnp.max(s_e, axis=1, keepdims=True),
                    jnp.max(s_o, axis=1, keepdims=True))
    z = (jnp.sum(jnp.exp(s_e - m), axis=1, keepdims=True)
         + jnp.sum(jnp.exp(s_o - m), axis=1, keepdims=True))
    ll = m + jnp.log(z)
    out_ref[:, :] = jnp.broadcast_to(-ll, (_B, 128))


def _run(logits, targets, loglen, tgtlen):
    return pl.pallas_call(
        _ctc_kernel,
        out_shape=jax.ShapeDtypeStruct((_B, 128), jnp.float32),
        scratch_shapes=[
            pltpu.VMEM((_B, _TB, _W), jnp.float32),
            pltpu.VMEM((_B, _TB, _W), jnp.float32),
            pltpu.VMEM((_B, _V, _W), jnp.float32),
        ],
    )(logits, targets, loglen, tgtlen)


@jax.jit
def kernel(logits, targets, logits_lengths, targets_lengths):
    loglen = logits_lengths.astype(jnp.int32).reshape(_B, 1)
    tgtlen = targets_lengths.astype(jnp.int32).reshape(_B, 1)
    out = _run(logits, targets.astype(jnp.int32), loglen, tgtlen)
    return out[:, 0]
